# baseline (speedup 1.0000x reference)
import functools

import jax
import jax.numpy as jnp
from jax import lax
from jax.experimental import pallas as pl
from jax.experimental.pallas import tpu as pltpu
from jax.experimental.pallas import tpu_sc as plsc

_B, _H, _W, _C = 8, 384, 384, 96
_MIN_HW = 384
_ROW = _W * _C
_NROWS = _B * _MIN_HW
_NC, _NS = 2, 16
_NW = _NC * _NS
_RPW = _NROWS // _NW
_STEP = 2.0 / float(_MIN_HW - 1)
_HALF = _MIN_HW * 0.5


def _floorf(x):
    t = x.astype(jnp.int32).astype(jnp.float32)
    return jnp.where(x < t, t - 1.0, t)


def _body(tpad_hbm, imgs_hbm, out_hbm,
          tvec, aref, bref, obuf, x0t, x1t, wx0t, wx1t):
    wid = lax.axis_index("s") * _NC + lax.axis_index("c")
    b = wid // (_NW // _B)
    seg = wid % (_NW // _B)
    lanes = lax.iota(jnp.int32, 16)

    pltpu.sync_copy(tpad_hbm, tvec)
    tv = tvec[pl.ds(3 * b, 16)]
    tx = tv[0]
    ty = tv[1]
    s = tv[2] + 1.0

    def build_tables(jb, carry):
        jv = (lanes + jb * 16).astype(jnp.float32)
        xv = ((jv * _STEP - 1.0) * s + tx + 1.0) * _HALF
        xf = _floorf(xv)
        x0f = jnp.clip(xf, 0.0, float(_W - 1))
        x1f = jnp.clip(xf + 1.0, 0.0, float(_W - 1))
        sl = pl.ds(jb * 16, 16)
        x0t[sl] = x0f.astype(jnp.int32) * _C
        x1t[sl] = x1f.astype(jnp.int32) * _C
        wx0t[sl] = x1f - xv
        wx1t[sl] = xv - x0f
        return carry

    lax.fori_loop(0, _W // 16, build_tables, 0)

    def row_body(t, carry):
        r = wid * _RPW + t
        i = seg * _RPW + t
        fi = i.astype(jnp.float32)
        y = ((fi * _STEP - 1.0) * s + ty + 1.0) * _HALF
        yf = _floorf(y)
        y0f = jnp.clip(yf, 0.0, float(_H - 1))
        y1f = jnp.clip(yf + 1.0, 0.0, float(_H - 1))
        wy0 = y1f - y
        wy1 = y - y0f
        y0i = y0f.astype(jnp.int32)
        y1i = y1f.astype(jnp.int32)
        base = b * _H
        pltpu.sync_copy(imgs_hbm.at[pl.ds((base + y0i) * _ROW, _ROW)], aref)
        pltpu.sync_copy(imgs_hbm.at[pl.ds((base + y1i) * _ROW, _ROW)], bref)

        def jb_body(jb, jcarry):
            sl = pl.ds(jb * 16, 16)
            x0v = x0t[sl]
            x1v = x1t[sl]
            wx0v = wx0t[sl]
            wx1v = wx1t[sl]
            wav = wx0v * wy0
            wbv = wx1v * wy0
            wcv = wx0v * wy1
            wdv = wx1v * wy1
            obase = (lanes + jb * 16) * _C
            for c in range(_C):
                ia = plsc.load_gather(aref, [x0v + c])
                ib = plsc.load_gather(aref, [x1v + c])
                ic = plsc.load_gather(bref, [x0v + c])
                idv = plsc.load_gather(bref, [x1v + c])
                o = wav * ia + wbv * ib + wcv * ic + wdv * idv
                plsc.store_scatter(obuf, [obase + c], o)
            return jcarry

        lax.fori_loop(0, _W // 16, jb_body, 0)
        pltpu.sync_copy(obuf, out_hbm.at[pl.ds(r * _ROW, _ROW)])
        return carry

    lax.fori_loop(0, _RPW, row_body, 0)


_mesh = plsc.VectorSubcoreMesh(
    core_axis_name="c", subcore_axis_name="s", num_cores=_NC, num_subcores=_NS)

_sc_bilinear = functools.partial(
    pl.kernel,
    out_type=jax.ShapeDtypeStruct((_NROWS * _ROW,), jnp.float32),
    mesh=_mesh,
    compiler_params=pltpu.CompilerParams(needs_layout_passes=False),
    scratch_types=[
        pltpu.VMEM((48,), jnp.float32),
        pltpu.VMEM((_ROW,), jnp.float32),
        pltpu.VMEM((_ROW,), jnp.float32),
        pltpu.VMEM((_ROW,), jnp.float32),
        pltpu.VMEM((_W,), jnp.int32),
        pltpu.VMEM((_W,), jnp.int32),
        pltpu.VMEM((_W,), jnp.float32),
        pltpu.VMEM((_W,), jnp.float32),
    ],
)(_body)


def kernel(transforms, imgs):
    tpad = jnp.zeros((48,), jnp.float32).at[:24].set(
        transforms.reshape(24).astype(jnp.float32))
    out = _sc_bilinear(tpad, imgs.reshape(-1))
    return out.reshape(_B, _MIN_HW, _MIN_HW, _C)


# speedup vs baseline: 2.4904x; 2.4904x over previous
import functools

import jax
import jax.numpy as jnp
from jax import lax
from jax.experimental import pallas as pl
from jax.experimental.pallas import tpu as pltpu
from jax.experimental.pallas import tpu_sc as plsc

_B, _H, _W, _C = 8, 384, 384, 96
_MIN_HW = 384
_ROW = _W * _C
_NROWS = _B * _MIN_HW
_NC, _NS = 2, 16
_NW = _NC * _NS
_RPW = _NROWS // _NW
_STEP = 2.0 / float(_MIN_HW - 1)
_HALF = _MIN_HW * 0.5


def _floorf(x):
    t = x.astype(jnp.int32).astype(jnp.float32)
    return jnp.where(x < t, t - 1.0, t)


def _body(tpad_hbm, imgs_hbm, out_hbm, tvec, aref, bref, obuf):
    wid = lax.axis_index("s") * _NC + lax.axis_index("c")
    b = wid // (_NW // _B)
    seg = wid % (_NW // _B)

    pltpu.sync_copy(tpad_hbm, tvec)
    tv = tvec[pl.ds(3 * b, 16)]
    tx = tv[0]
    ty = tv[1]
    s = tv[2] + 1.0

    def row_body(t, carry):
        r = wid * _RPW + t
        i = seg * _RPW + t
        fi = i.astype(jnp.float32)
        y = ((fi * _STEP - 1.0) * s + ty + 1.0) * _HALF
        yf = _floorf(y)
        y0f = jnp.clip(yf, 0.0, float(_H - 1))
        y1f = jnp.clip(yf + 1.0, 0.0, float(_H - 1))
        wy0 = y1f - y
        wy1 = y - y0f
        y0i = y0f.astype(jnp.int32)
        y1i = y1f.astype(jnp.int32)
        base = b * _H
        pltpu.sync_copy(imgs_hbm.at[pl.ds((base + y0i) * _ROW, _ROW)], aref)
        pltpu.sync_copy(imgs_hbm.at[pl.ds((base + y1i) * _ROW, _ROW)], bref)

        def j_body(j, jcarry):
            fj = j.astype(jnp.float32)
            xs = ((fj * _STEP - 1.0) * s + tx + 1.0) * _HALF
            xf = _floorf(xs)
            x0f = jnp.clip(xf, 0.0, float(_W - 1))
            x1f = jnp.clip(xf + 1.0, 0.0, float(_W - 1))
            base0 = x0f.astype(jnp.int32) * _C
            base1 = x1f.astype(jnp.int32) * _C
            wx0 = x1f - xs
            wx1 = xs - x0f
            wav = jnp.full((16,), wx0 * wy0, jnp.float32)
            wbv = jnp.full((16,), wx1 * wy0, jnp.float32)
            wcv = jnp.full((16,), wx0 * wy1, jnp.float32)
            wdv = jnp.full((16,), wx1 * wy1, jnp.float32)
            obase = j * _C
            for c in range(0, _C, 16):
                va = aref[pl.ds(base0 + c, 16)]
                vb = aref[pl.ds(base1 + c, 16)]
                vc = bref[pl.ds(base0 + c, 16)]
                vd = bref[pl.ds(base1 + c, 16)]
                obuf[pl.ds(obase + c, 16)] = (
                    wav * va + wbv * vb + wcv * vc + wdv * vd)
            return jcarry

        lax.fori_loop(0, _W, j_body, 0)
        pltpu.sync_copy(obuf, out_hbm.at[pl.ds(r * _ROW, _ROW)])
        return carry

    lax.fori_loop(0, _RPW, row_body, 0)


_mesh = plsc.VectorSubcoreMesh(
    core_axis_name="c", subcore_axis_name="s", num_cores=_NC, num_subcores=_NS)

_sc_bilinear = functools.partial(
    pl.kernel,
    out_type=jax.ShapeDtypeStruct((_NROWS * _ROW,), jnp.float32),
    mesh=_mesh,
    compiler_params=pltpu.CompilerParams(needs_layout_passes=False),
    scratch_types=[
        pltpu.VMEM((48,), jnp.float32),
        pltpu.VMEM((_ROW,), jnp.float32),
        pltpu.VMEM((_ROW,), jnp.float32),
        pltpu.VMEM((_ROW,), jnp.float32),
    ],
)(_body)


def kernel(transforms, imgs):
    tpad = jnp.zeros((48,), jnp.float32).at[:24].set(
        transforms.reshape(24).astype(jnp.float32))
    out = _sc_bilinear(tpad, imgs.reshape(-1))
    return out.reshape(_B, _MIN_HW, _MIN_HW, _C)


# speedup vs baseline: 2.8513x; 1.1449x over previous
import functools

import jax
import jax.numpy as jnp
from jax import lax
from jax.experimental import pallas as pl
from jax.experimental.pallas import tpu as pltpu
from jax.experimental.pallas import tpu_sc as plsc

_B, _H, _W, _C = 8, 384, 384, 96
_MIN_HW = 384
_HC = _C // 2
_NROWS = _B * _MIN_HW
_NC, _NS = 2, 16
_NW = _NC * _NS
_RPW = _NROWS // _NW
_STEP = 2.0 / float(_MIN_HW - 1)
_HALF = _MIN_HW * 0.5
_JU = 4


def _floorf(x):
    t = x.astype(jnp.int32).astype(jnp.float32)
    return jnp.where(x < t, t - 1.0, t)


def _body(tpad_hbm, imgs_hbm, out_hbm, tvec,
          a0, b0, o0, a1, b1, o1, si0, si1, so0, so1):
    wid = lax.axis_index("s") * _NC + lax.axis_index("c")
    b = wid // (_NW // _B)
    seg = wid % (_NW // _B)

    pltpu.sync_copy(tpad_hbm, tvec)
    tv = tvec[pl.ds(3 * b, 16)]
    tx = tv[0]
    ty = tv[1]
    s = tv[2] + 1.0

    ins = ((a0, b0, si0), (a1, b1, si1))
    outs = ((o0, so0), (o1, so1))

    def row_scalars(t):
        fi = (seg * _RPW + t).astype(jnp.float32)
        y = ((fi * _STEP - 1.0) * s + ty + 1.0) * _HALF
        yf = _floorf(y)
        y0f = jnp.clip(yf, 0.0, float(_H - 1))
        y1f = jnp.clip(yf + 1.0, 0.0, float(_H - 1))
        return (y0f.astype(jnp.int32), y1f.astype(jnp.int32),
                y1f - y, y - y0f)

    def issue_in(y0i, y1i, h):
        aref, bref, sem = ins[h]
        base = b * _H
        c0 = _HC * h
        pltpu.async_copy(
            imgs_hbm.at[pl.ds((base + y0i) * _W, _W), pl.ds(c0, _HC)],
            aref, sem)
        pltpu.async_copy(
            imgs_hbm.at[pl.ds((base + y1i) * _W, _W), pl.ds(c0, _HC)],
            bref, sem)

    def wait_in(h):
        aref, bref, sem = ins[h]
        src = imgs_hbm.at[pl.ds(0, _W), pl.ds(0, _HC)]
        pltpu.make_async_copy(src, aref, sem).wait()
        pltpu.make_async_copy(src, bref, sem).wait()

    def issue_out(t, h):
        obuf, sem = outs[h]
        r = wid * _RPW + t
        pltpu.async_copy(
            obuf, out_hbm.at[pl.ds(r * _W, _W), pl.ds(_HC * h, _HC)], sem)

    def wait_out(h):
        obuf, sem = outs[h]
        dst = out_hbm.at[pl.ds(0, _W), pl.ds(0, _HC)]
        pltpu.make_async_copy(obuf, dst, sem).wait()

    def compute_half(wy0, wy1, h):
        aref, bref, sem_unused = ins[h]
        obuf = outs[h][0]

        def j_block(jb, jcarry):
            for k in range(_JU):
                j = jb * _JU + k
                fj = j.astype(jnp.float32)
                xs = ((fj * _STEP - 1.0) * s + tx + 1.0) * _HALF
                xf = _floorf(xs)
                x0f = jnp.clip(xf, 0.0, float(_W - 1))
                x1f = jnp.clip(xf + 1.0, 0.0, float(_W - 1))
                r0 = x0f.astype(jnp.int32)
                r1 = x1f.astype(jnp.int32)
                wx0 = x1f - xs
                wx1 = xs - x0f
                wav = jnp.full((16,), wx0 * wy0, jnp.float32)
                wbv = jnp.full((16,), wx1 * wy0, jnp.float32)
                wcv = jnp.full((16,), wx0 * wy1, jnp.float32)
                wdv = jnp.full((16,), wx1 * wy1, jnp.float32)
                for c in range(0, _HC, 16):
                    sl = pl.ds(c, 16)
                    va = aref[r0, sl]
                    vb = aref[r1, sl]
                    vc = bref[r0, sl]
                    vd = bref[r1, sl]
                    obuf[j, sl] = wav * va + wbv * vb + wcv * vc + wdv * vd
            return jcarry

        lax.fori_loop(0, _W // _JU, j_block, 0)

    y0i, y1i, wy0, wy1 = row_scalars(jnp.int32(0))
    issue_in(y0i, y1i, 0)
    issue_in(y0i, y1i, 1)

    wait_in(0)
    compute_half(wy0, wy1, 0)
    issue_out(jnp.int32(0), 0)
    ny0, ny1, _, _ = row_scalars(jnp.int32(1))
    issue_in(ny0, ny1, 0)
    wait_in(1)
    compute_half(wy0, wy1, 1)
    issue_out(jnp.int32(0), 1)
    issue_in(ny0, ny1, 1)

    def row_body(t, carry):
        _, _, wy0, wy1 = row_scalars(t)
        ny0, ny1, _, _ = row_scalars(t + 1)
        wait_in(0)
        wait_out(0)
        compute_half(wy0, wy1, 0)
        issue_out(t, 0)
        issue_in(ny0, ny1, 0)
        wait_in(1)
        wait_out(1)
        compute_half(wy0, wy1, 1)
        issue_out(t, 1)
        issue_in(ny0, ny1, 1)
        return carry

    lax.fori_loop(1, _RPW, row_body, 0)

    wait_in(0)
    wait_in(1)
    wait_out(0)
    wait_out(1)


_mesh = plsc.VectorSubcoreMesh(
    core_axis_name="c", subcore_axis_name="s", num_cores=_NC, num_subcores=_NS)

_sc_bilinear = functools.partial(
    pl.kernel,
    out_type=jax.ShapeDtypeStruct((_NROWS * _W, _C), jnp.float32),
    mesh=_mesh,
    compiler_params=pltpu.CompilerParams(
        needs_layout_passes=False, use_tc_tiling_on_sc=False),
    scratch_types=[
        pltpu.VMEM((48,), jnp.float32),
        pltpu.VMEM((_W, _HC), jnp.float32),
        pltpu.VMEM((_W, _HC), jnp.float32),
        pltpu.VMEM((_W, _HC), jnp.float32),
        pltpu.VMEM((_W, _HC), jnp.float32),
        pltpu.VMEM((_W, _HC), jnp.float32),
        pltpu.VMEM((_W, _HC), jnp.float32),
        pltpu.SemaphoreType.DMA,
        pltpu.SemaphoreType.DMA,
        pltpu.SemaphoreType.DMA,
        pltpu.SemaphoreType.DMA,
    ],
)(_body)


def kernel(transforms, imgs):
    tpad = jnp.zeros((48,), jnp.float32).at[:24].set(
        transforms.reshape(24).astype(jnp.float32))
    out = _sc_bilinear(tpad, imgs.reshape(_B * _H * _W, _C))
    return out.reshape(_B, _MIN_HW, _MIN_HW, _C)


# speedup vs baseline: 10.4134x; 3.6522x over previous
import functools

import jax
import jax.numpy as jnp
from jax import lax
from jax.experimental import pallas as pl
from jax.experimental.pallas import tpu as pltpu
from jax.experimental.pallas import tpu_sc as plsc

_B, _H, _W, _C = 8, 384, 384, 96
_MIN_HW = 384
_HC = _C // 2
_NROWS = _B * _MIN_HW
_NC, _NS = 2, 16
_NW = _NC * _NS
_TPW = _NROWS // _NW
_TPB = _TPW // _B
_NB = _W // 16
_STEP = 2.0 / float(_MIN_HW - 1)
_HALF = _MIN_HW * 0.5
_SI = 5
_SF = 4


def _floorf(x):
    t = x.astype(jnp.int32).astype(jnp.float32)
    return jnp.where(x < t, t - 1.0, t)


def _body(tpad_hbm, imgs_hbm, out_hbm, tvec,
          a0, b0, o0, a1, b1, o1, smi, smf, si0, si1, so0, so1):
    wid = lax.axis_index("s") * _NC + lax.axis_index("c")
    lanes = lax.iota(jnp.int32, 16)
    zv = jnp.zeros((16,), jnp.float32)

    pltpu.sync_copy(tpad_hbm, tvec)

    for bb in range(_B):
        tvb = tvec[pl.ds(3 * bb, 16)]
        txb = tvb[0]
        tyb = tvb[1]
        sb = tvb[2] + 1.0
        xa = sb * jnp.float32(_STEP * _HALF)
        xc = (txb + 1.0 - sb) * jnp.float32(_HALF)

        def _cnt(j, acc):
            xs = xa * j.astype(jnp.float32) + xc
            return (acc[0] + jnp.where(xs < 0.0, 1, 0),
                    acc[1] + jnp.where(xs >= float(_W - 1), 1, 0))

        nlo, nhi = lax.fori_loop(0, _W, _cnt, (jnp.int32(0), jnp.int32(0)))
        jlo = nlo
        jhi = jnp.maximum(_W - nhi, jlo)
        kblo = jlo >> 4
        kbhi = (jhi + 15) >> 4

        def _task(d, carry):
            t = bb * _TPB + d
            i = d * _NW + wid
            fi = i.astype(jnp.float32)
            y = ((fi * _STEP - 1.0) * sb + tyb + 1.0) * _HALF
            yf = _floorf(y)
            y0f = jnp.clip(yf, 0.0, float(_H - 1))
            y1f = jnp.clip(yf + 1.0, 0.0, float(_H - 1))
            y0i = y0f.astype(jnp.int32)
            y1i = y1f.astype(jnp.int32)
            live = y0i != y1i
            base = bb * _H
            smi[_SI * t] = (base + y0i) * _C
            smi[_SI * t + 1] = (base + y1i) * _C
            smi[_SI * t + 2] = (base + i) * _C
            smi[_SI * t + 3] = jnp.where(live, kblo, 0)
            smi[_SI * t + 4] = jnp.where(live, kbhi, 0)
            smf[_SF * t] = y1f - y
            smf[_SF * t + 1] = y - y0f
            smf[_SF * t + 2] = xa
            smf[_SF * t + 3] = xc
            return carry

        lax.fori_loop(0, _TPB, _task, 0)

    smi[_SI * _TPW] = 0
    smi[_SI * _TPW + 1] = 0
    smi[_SI * _TPW + 2] = 0
    smi[_SI * _TPW + 3] = 0
    smi[_SI * _TPW + 4] = 0

    ins = ((a0, b0, si0), (a1, b1, si1))
    outs = ((o0, so0), (o1, so1))

    def issue_in(ain0, ain1, h):
        aref, bref, sem = ins[h]
        s0 = pl.multiple_of(ain0 + _HC * h, 8)
        s1 = pl.multiple_of(ain1 + _HC * h, 8)
        pltpu.async_copy(imgs_hbm.at[pl.ds(s0, _HC)], aref, sem)
        pltpu.async_copy(imgs_hbm.at[pl.ds(s1, _HC)], bref, sem)

    def wait_in(h):
        aref, bref, sem = ins[h]
        src = imgs_hbm.at[pl.ds(0, _HC)]
        pltpu.make_async_copy(src, aref, sem).wait()
        pltpu.make_async_copy(src, bref, sem).wait()

    def issue_out(obase, h):
        obuf, sem = outs[h]
        s0 = pl.multiple_of(obase + _HC * h, 8)
        pltpu.async_copy(obuf, out_hbm.at[pl.ds(s0, _HC)], sem)

    def wait_out(h):
        obuf, sem = outs[h]
        pltpu.make_async_copy(obuf, out_hbm.at[pl.ds(0, _HC)], sem).wait()

    def prefetch(tn):
        nain0 = smi[_SI * tn]
        nain1 = smi[_SI * tn + 1]

        @pl.when(nain0 != nain1)
        def _():
            issue_in(nain0, nain1, 0)
            issue_in(nain0, nain1, 1)

    def compute_half(t, h):
        aref, bref, _ = ins[h]
        obuf = outs[h][0]
        kblo = smi[_SI * t + 3]
        kbhi = smi[_SI * t + 4]
        wy0 = smf[_SF * t]
        wy1 = smf[_SF * t + 1]
        xa = smf[_SF * t + 2]
        xc = smf[_SF * t + 3]

        def zblock(kb, carry):
            sl = pl.ds(kb * 16, 16)
            for c in range(_HC):
                obuf[c, sl] = zv
            return carry

        def mblock(kb, carry):
            jv = (lanes + kb * 16).astype(jnp.float32)
            xs = xa * jv + xc
            xf = _floorf(xs)
            x0f = jnp.clip(xf, 0.0, float(_W - 1))
            x1f = jnp.clip(xf + 1.0, 0.0, float(_W - 1))
            x0v = x0f.astype(jnp.int32)
            x1v = x1f.astype(jnp.int32)
            wx0 = x1f - xs
            wx1 = xs - x0f
            wav = wx0 * wy0
            wbv = wx1 * wy0
            wcv = wx0 * wy1
            wdv = wx1 * wy1
            sl = pl.ds(kb * 16, 16)
            for c in range(_HC):
                cv = jnp.full((16,), c, jnp.int32)
                va = plsc.load_gather(aref, [cv, x0v])
                vb = plsc.load_gather(aref, [cv, x1v])
                vc = plsc.load_gather(bref, [cv, x0v])
                vd = plsc.load_gather(bref, [cv, x1v])
                obuf[c, sl] = wav * va + wbv * vb + wcv * vc + wdv * vd
            return carry

        lax.fori_loop(0, kblo, zblock, 0)
        lax.fori_loop(kblo, kbhi, mblock, 0)
        lax.fori_loop(kbhi, _NB, zblock, 0)

    def do_task(t, first):
        ain0 = smi[_SI * t]
        ain1 = smi[_SI * t + 1]
        obase = smi[_SI * t + 2]
        live = ain0 != ain1

        @pl.when(live)
        def _():
            wait_in(0)
        if not first:
            wait_out(0)
        compute_half(t, 0)
        issue_out(obase, 0)

        @pl.when(live)
        def _():
            wait_in(1)
        if not first:
            wait_out(1)
        compute_half(t, 1)
        issue_out(obase, 1)
        prefetch(t + 1)

    prefetch(jnp.int32(0))
    do_task(jnp.int32(0), True)

    def task_body(t, carry):
        do_task(t, False)
        return carry

    lax.fori_loop(1, _TPW, task_body, 0)
    wait_out(0)
    wait_out(1)


_mesh = plsc.VectorSubcoreMesh(
    core_axis_name="c", subcore_axis_name="s", num_cores=_NC, num_subcores=_NS)

_sc_bilinear = functools.partial(
    pl.kernel,
    out_type=jax.ShapeDtypeStruct((_B * _H * _C, _W), jnp.float32),
    mesh=_mesh,
    compiler_params=pltpu.CompilerParams(needs_layout_passes=False),
    scratch_types=[
        pltpu.VMEM((48,), jnp.float32),
        pltpu.VMEM((_HC, _W), jnp.float32),
        pltpu.VMEM((_HC, _W), jnp.float32),
        pltpu.VMEM((_HC, _W), jnp.float32),
        pltpu.VMEM((_HC, _W), jnp.float32),
        pltpu.VMEM((_HC, _W), jnp.float32),
        pltpu.VMEM((_HC, _W), jnp.float32),
        pltpu.SMEM((_SI * (_TPW + 1),), jnp.int32),
        pltpu.SMEM((_SF * _TPW,), jnp.float32),
        pltpu.SemaphoreType.DMA,
        pltpu.SemaphoreType.DMA,
        pltpu.SemaphoreType.DMA,
        pltpu.SemaphoreType.DMA,
    ],
)(_body)


def kernel(transforms, imgs):
    tpad = jnp.zeros((48,), jnp.float32).at[:24].set(
        transforms.reshape(24).astype(jnp.float32))
    imgs_t = imgs.transpose(0, 1, 3, 2).reshape(_B * _H * _C, _W)
    out = _sc_bilinear(tpad, imgs_t)
    return out.reshape(_B, _MIN_HW, _C, _MIN_HW).transpose(0, 1, 3, 2)


# speedup vs baseline: 19.1225x; 1.8363x over previous
import functools

import jax
import jax.numpy as jnp
from jax import lax
from jax.experimental import pallas as pl
from jax.experimental.pallas import tpu as pltpu
from jax.experimental.pallas import tpu_sc as plsc

_B, _H, _W, _C = 8, 384, 384, 96
_MIN_HW = 384
_HC = _C // 2
_NROWS = _B * _MIN_HW
_NC, _NS = 2, 16
_NW = _NC * _NS
_TPW = _NROWS // _NW
_TPB = _TPW // _B
_NB = _W // 16
_STEP = 2.0 / float(_MIN_HW - 1)
_HALF = _MIN_HW * 0.5
_SI = 5
_SF = 4


def _floorf(x):
    t = x.astype(jnp.int32).astype(jnp.float32)
    return jnp.where(x < t, t - 1.0, t)


def _body(tpad_hbm, imgs_hbm, out_hbm, tvec,
          a0, b0, o0, a1, b1, o1, smi, smf, si0, si1, so0, so1):
    wid = lax.axis_index("s") * _NC + lax.axis_index("c")
    lanes = lax.iota(jnp.int32, 16)
    zv = jnp.zeros((16,), jnp.float32)

    pltpu.sync_copy(tpad_hbm, tvec)

    for bb in range(_B):
        tvb = tvec[pl.ds(3 * bb, 16)]
        txb = tvb[0]
        tyb = tvb[1]
        sb = tvb[2] + 1.0
        xa = sb * jnp.float32(_STEP * _HALF)
        xc = (txb + 1.0 - sb) * jnp.float32(_HALF)

        def _cnt(j, acc):
            xs = xa * j.astype(jnp.float32) + xc
            return (acc[0] + jnp.where(xs < 0.0, 1, 0),
                    acc[1] + jnp.where(xs >= float(_W - 1), 1, 0))

        nlo, nhi = lax.fori_loop(0, _W, _cnt, (jnp.int32(0), jnp.int32(0)))
        jlo = nlo
        jhi = jnp.maximum(_W - nhi, jlo)
        kblo = jlo >> 4
        kbhi = (jhi + 15) >> 4

        def _task(d, carry):
            t = bb * _TPB + d
            i = d * _NW + wid
            fi = i.astype(jnp.float32)
            y = ((fi * _STEP - 1.0) * sb + tyb + 1.0) * _HALF
            yf = _floorf(y)
            y0f = jnp.clip(yf, 0.0, float(_H - 1))
            y1f = jnp.clip(yf + 1.0, 0.0, float(_H - 1))
            y0i = y0f.astype(jnp.int32)
            y1i = y1f.astype(jnp.int32)
            live = y0i != y1i
            base = bb * _H
            smi[_SI * t] = (base + y0i) * _C
            smi[_SI * t + 1] = (base + y1i) * _C
            smi[_SI * t + 2] = (base + i) * _C
            smi[_SI * t + 3] = jnp.where(live, kblo, 0)
            smi[_SI * t + 4] = jnp.where(live, kbhi, 0)
            smf[_SF * t] = y1f - y
            smf[_SF * t + 1] = y - y0f
            smf[_SF * t + 2] = xa
            smf[_SF * t + 3] = xc
            return carry

        lax.fori_loop(0, _TPB, _task, 0)

    smi[_SI * _TPW] = 0
    smi[_SI * _TPW + 1] = 0
    smi[_SI * _TPW + 2] = 0
    smi[_SI * _TPW + 3] = 0
    smi[_SI * _TPW + 4] = 0

    ins = ((a0, b0, si0), (a1, b1, si1))
    outs = ((o0, so0), (o1, so1))

    def issue_in(ain0, ain1, h):
        aref, bref, sem = ins[h]
        s0 = pl.multiple_of(ain0 + _HC * h, 8)
        s1 = pl.multiple_of(ain1 + _HC * h, 8)
        pltpu.async_copy(imgs_hbm.at[pl.ds(s0, _HC)], aref, sem)
        pltpu.async_copy(imgs_hbm.at[pl.ds(s1, _HC)], bref, sem)

    def wait_in(h):
        aref, bref, sem = ins[h]
        src = imgs_hbm.at[pl.ds(0, _HC)]
        pltpu.make_async_copy(src, aref, sem).wait()
        pltpu.make_async_copy(src, bref, sem).wait()

    def issue_out(obase, h):
        obuf, sem = outs[h]
        s0 = pl.multiple_of(obase + _HC * h, 8)
        pltpu.async_copy(obuf, out_hbm.at[pl.ds(s0, _HC)], sem)

    def wait_out(h):
        obuf, sem = outs[h]
        pltpu.make_async_copy(obuf, out_hbm.at[pl.ds(0, _HC)], sem).wait()

    def prefetch(tn):
        nain0 = smi[_SI * tn]
        nain1 = smi[_SI * tn + 1]

        @pl.when(nain0 != nain1)
        def _():
            issue_in(nain0, nain1, 0)
            issue_in(nain0, nain1, 1)

    def compute_half(t, h):
        aref, bref, _ = ins[h]
        obuf = outs[h][0]
        kblo = smi[_SI * t + 3]
        kbhi = smi[_SI * t + 4]
        wy0 = smf[_SF * t]
        wy1 = smf[_SF * t + 1]
        xa = smf[_SF * t + 2]
        xc = smf[_SF * t + 3]

        def zblock(kb, carry):
            sl = pl.ds(kb * 16, 16)
            for c in range(_HC):
                obuf[c, sl] = zv
            return carry

        def mblock(kb):
            jv = (lanes + kb * 16).astype(jnp.float32)
            xs = xa * jv + xc
            xf = _floorf(xs)
            x0f = jnp.clip(xf, 0.0, float(_W - 1))
            x1f = jnp.clip(xf + 1.0, 0.0, float(_W - 1))
            x0v = x0f.astype(jnp.int32)
            x1v = x1f.astype(jnp.int32)
            wx0 = x1f - xs
            wx1 = xs - x0f
            wav = wx0 * wy0
            wbv = wx1 * wy0
            wcv = wx0 * wy1
            wdv = wx1 * wy1
            sl = pl.ds(kb * 16, 16)

            @plsc.parallel_loop(0, _HC, step=1, unroll=8)
            def _cloop(c):
                cv = jnp.full((16,), c, jnp.int32)
                va = plsc.load_gather(aref, [cv, x0v])
                vb = plsc.load_gather(aref, [cv, x1v])
                vc = plsc.load_gather(bref, [cv, x0v])
                vd = plsc.load_gather(bref, [cv, x1v])
                obuf[c, sl] = (wav * va + wbv * vb) + (wcv * vc + wdv * vd)

        def mloop(kb, carry):
            mblock(kb)
            return carry

        lax.fori_loop(0, kblo, zblock, 0)
        lax.fori_loop(kblo, kbhi, mloop, 0)
        lax.fori_loop(kbhi, _NB, zblock, 0)

    def do_task(t, first):
        ain0 = smi[_SI * t]
        ain1 = smi[_SI * t + 1]
        obase = smi[_SI * t + 2]
        live = ain0 != ain1

        @pl.when(live)
        def _():
            wait_in(0)
        if not first:
            wait_out(0)
        compute_half(t, 0)
        issue_out(obase, 0)

        @pl.when(live)
        def _():
            wait_in(1)
        if not first:
            wait_out(1)
        compute_half(t, 1)
        issue_out(obase, 1)
        prefetch(t + 1)

    prefetch(jnp.int32(0))
    do_task(jnp.int32(0), True)

    def task_body(t, carry):
        do_task(t, False)
        return carry

    lax.fori_loop(1, _TPW, task_body, 0)
    wait_out(0)
    wait_out(1)


_mesh = plsc.VectorSubcoreMesh(
    core_axis_name="c", subcore_axis_name="s", num_cores=_NC, num_subcores=_NS)

_sc_bilinear = functools.partial(
    pl.kernel,
    out_type=jax.ShapeDtypeStruct((_B * _H * _C, _W), jnp.float32),
    mesh=_mesh,
    compiler_params=pltpu.CompilerParams(needs_layout_passes=False),
    scratch_types=[
        pltpu.VMEM((48,), jnp.float32),
        pltpu.VMEM((_HC, _W), jnp.float32),
        pltpu.VMEM((_HC, _W), jnp.float32),
        pltpu.VMEM((_HC, _W), jnp.float32),
        pltpu.VMEM((_HC, _W), jnp.float32),
        pltpu.VMEM((_HC, _W), jnp.float32),
        pltpu.VMEM((_HC, _W), jnp.float32),
        pltpu.SMEM((_SI * (_TPW + 1),), jnp.int32),
        pltpu.SMEM((_SF * _TPW,), jnp.float32),
        pltpu.SemaphoreType.DMA,
        pltpu.SemaphoreType.DMA,
        pltpu.SemaphoreType.DMA,
        pltpu.SemaphoreType.DMA,
    ],
)(_body)


def kernel(transforms, imgs):
    tpad = jnp.zeros((48,), jnp.float32).at[:24].set(
        transforms.reshape(24).astype(jnp.float32))
    imgs_t = imgs.transpose(0, 1, 3, 2).reshape(_B * _H * _C, _W)
    out = _sc_bilinear(tpad, imgs_t)
    return out.reshape(_B, _MIN_HW, _C, _MIN_HW).transpose(0, 1, 3, 2)
